# Initial kernel scaffold; baseline (speedup 1.0000x reference)
import functools

import jax
import jax.numpy as jnp
from jax import lax
from jax.experimental import pallas as pl
from jax.experimental.pallas import tpu as pltpu
from jax.experimental.pallas import tpu_sc as plsc

N = 10000
E = 320000
D = 128

NC = 2
NS = 16
NW = NC * NS
C = 128
NCH = E // C
BASE_CH = NCH // NW
EXTRA = NCH - BASE_CH * NW
ROWS_PER_TILE = N // NS

_mesh = plsc.VectorSubcoreMesh(core_axis_name="c", subcore_axis_name="s")


@functools.partial(
    pl.kernel,
    mesh=_mesh,
    out_type=jax.ShapeDtypeStruct((NC, N, 16), jnp.float32),
    scratch_types=[
        pltpu.VMEM((C,), jnp.int32),
        pltpu.VMEM((C, 16), jnp.float32),
        pltpu.VMEM_SHARED((N, 16), jnp.float32),
    ],
)
def _deg_kernel(dst_hbm, ones_hbm, zeros_hbm, out_hbm, idx_v, ones_v, accum):
    c = lax.axis_index("c")
    s = lax.axis_index("s")
    wid = s * NC + c
    r0 = s * ROWS_PER_TILE
    pltpu.sync_copy(zeros_hbm.at[pl.ds(r0, ROWS_PER_TILE)],
                    accum.at[pl.ds(r0, ROWS_PER_TILE)])
    pltpu.sync_copy(ones_hbm, ones_v)
    plsc.subcore_barrier()

    n_my = jnp.where(wid < EXTRA, BASE_CH + 1, BASE_CH)

    def body(k, carry):
        ch = wid + k * NW
        pltpu.sync_copy(dst_hbm.at[pl.ds(ch * C, C)], idx_v)
        pltpu.sync_copy(ones_v, accum.at[idx_v], add=True)
        return carry

    lax.fori_loop(0, n_my, body, 0)
    plsc.subcore_barrier()
    pltpu.sync_copy(accum.at[pl.ds(r0, ROWS_PER_TILE)],
                    out_hbm.at[c, pl.ds(r0, ROWS_PER_TILE)])


@functools.partial(
    pl.kernel,
    mesh=_mesh,
    out_type=jax.ShapeDtypeStruct((NC, N, D), jnp.float32),
    scratch_types=[
        pltpu.VMEM((C,), jnp.int32),
        pltpu.VMEM((C,), jnp.int32),
        pltpu.VMEM((C, D), jnp.float32),
        pltpu.VMEM_SHARED((N, D), jnp.float32),
        pltpu.SemaphoreType.DMA,
    ],
)
def _spmm_kernel(g_hbm, src_hbm, dst_hbm, zeros_hbm, out_hbm,
                 src_v, dst_v, rows_v, accum, sem):
    c = lax.axis_index("c")
    s = lax.axis_index("s")
    wid = s * NC + c
    r0 = s * ROWS_PER_TILE
    pltpu.sync_copy(zeros_hbm.at[pl.ds(r0, ROWS_PER_TILE)],
                    accum.at[pl.ds(r0, ROWS_PER_TILE)])
    plsc.subcore_barrier()

    n_my = jnp.where(wid < EXTRA, BASE_CH + 1, BASE_CH)

    def body(k, carry):
        ch = wid + k * NW
        pltpu.sync_copy(src_hbm.at[pl.ds(ch * C, C)], src_v)
        pltpu.sync_copy(dst_hbm.at[pl.ds(ch * C, C)], dst_v)
        pltpu.async_copy(g_hbm.at[src_v], rows_v, sem).wait()
        pltpu.sync_copy(rows_v, accum.at[dst_v], add=True)
        return carry

    lax.fori_loop(0, n_my, body, 0)
    plsc.subcore_barrier()
    pltpu.sync_copy(accum.at[pl.ds(r0, ROWS_PER_TILE)],
                    out_hbm.at[c, pl.ds(r0, ROWS_PER_TILE)])


NB = 1000
GRID = N // NB


def _first_body(p_ref, x_ref, w_ref, g_ref, u_ref):
    p = p_ref[...]
    degsum = jnp.sum(p[0] + p[1], axis=-1, keepdims=True)
    deg = degsum * (1.0 / 16.0) + 1.0
    u = lax.rsqrt(deg)
    u_ref[...] = jnp.broadcast_to(u, (NB, 16))
    h = jnp.dot(x_ref[...], w_ref[...], preferred_element_type=jnp.float32)
    g_ref[...] = h * u


def _mid_body(s_ref, g_ref, u_ref, b_ref, w_ref, o_ref):
    sv = s_ref[...]
    u = u_ref[...][:, :1]
    t = (sv[0] + sv[1] + g_ref[...]) * u + b_ref[...]
    z = jnp.maximum(t, 0.0)
    o_ref[...] = jnp.dot(z, w_ref[...], preferred_element_type=jnp.float32) * u


def _last_body(s_ref, g_ref, u_ref, b_ref, o_ref):
    sv = s_ref[...]
    u = u_ref[...][:, :1]
    o_ref[...] = (sv[0] + sv[1] + g_ref[...]) * u + b_ref[...]


_spec_p = pl.BlockSpec((2, NB, 16), lambda i: (0, i, 0))
_spec_x = pl.BlockSpec((NB, D), lambda i: (i, 0))
_spec_w = pl.BlockSpec((D, D), lambda i: (0, 0))
_spec_s = pl.BlockSpec((2, NB, D), lambda i: (0, i, 0))
_spec_u = pl.BlockSpec((NB, 16), lambda i: (i, 0))
_spec_b = pl.BlockSpec((1, D), lambda i: (0, 0))

_first_tc = pl.pallas_call(
    _first_body,
    grid=(GRID,),
    in_specs=[_spec_p, _spec_x, _spec_w],
    out_specs=[_spec_x, _spec_u],
    out_shape=[jax.ShapeDtypeStruct((N, D), jnp.float32),
               jax.ShapeDtypeStruct((N, 16), jnp.float32)],
)

_mid_tc = pl.pallas_call(
    _mid_body,
    grid=(GRID,),
    in_specs=[_spec_s, _spec_x, _spec_u, _spec_b, _spec_w],
    out_specs=_spec_x,
    out_shape=jax.ShapeDtypeStruct((N, D), jnp.float32),
)

_last_tc = pl.pallas_call(
    _last_body,
    grid=(GRID,),
    in_specs=[_spec_s, _spec_x, _spec_u, _spec_b],
    out_specs=_spec_x,
    out_shape=jax.ShapeDtypeStruct((N, D), jnp.float32),
)


@jax.jit
def kernel(x, adj_t, W1, b1, W2, b2, W3, b3):
    adj = adj_t.astype(jnp.int32)
    src = adj[0]
    dst = adj[1]
    ones16 = jnp.ones((C, 16), jnp.float32)
    zeros16 = jnp.zeros((N, 16), jnp.float32)
    zerosND = jnp.zeros((N, D), jnp.float32)

    p = _deg_kernel(dst, ones16, zeros16)
    g1, u16 = _first_tc(p, x, W1)
    s1 = _spmm_kernel(g1, src, dst, zerosND)
    g2 = _mid_tc(s1, g1, u16, b1.reshape(1, D), W2)
    s2 = _spmm_kernel(g2, src, dst, zerosND)
    g3 = _mid_tc(s2, g2, u16, b2.reshape(1, D), W3)
    s3 = _spmm_kernel(g3, src, dst, zerosND)
    out = _last_tc(s3, g3, u16, b3.reshape(1, D))
    return out


# speedup vs baseline: 13.0050x; 13.0050x over previous
import functools

import jax
import jax.numpy as jnp
from jax import lax
from jax.experimental import pallas as pl
from jax.experimental.pallas import tpu as pltpu
from jax.experimental.pallas import tpu_sc as plsc

N = 10000
E = 320000
D = 128

NC = 2
NS = 16
NW = NC * NS
C = 128
NCH = E // C
BASE_CH = NCH // NW
EXTRA = NCH - BASE_CH * NW
ROWS_PER_TILE = 624
TAIL_R0 = ROWS_PER_TILE * NS
TAIL_ROWS = N - TAIL_R0


def _copy_rows(copy_fn, s):
    copy_fn(s * ROWS_PER_TILE, ROWS_PER_TILE)

    @pl.when(s == NS - 1)
    def _():
        copy_fn(TAIL_R0, TAIL_ROWS)

_mesh = plsc.VectorSubcoreMesh(core_axis_name="c", subcore_axis_name="s")


@functools.partial(
    pl.kernel,
    mesh=_mesh,
    out_type=jax.ShapeDtypeStruct((NC, N, D), jnp.float32),
    scratch_types=[
        pltpu.VMEM((C,), jnp.int32),
        pltpu.VMEM((C, D), jnp.float32),
        pltpu.VMEM_SHARED((N, D), jnp.float32),
    ],
)
def _deg_kernel(dst_hbm, ones_hbm, zeros_hbm, out_hbm, idx_v, ones_v, accum):
    c = lax.axis_index("c")
    s = lax.axis_index("s")
    wid = s * NC + c
    _copy_rows(lambda r0, nr: pltpu.sync_copy(
        zeros_hbm.at[pl.ds(r0, nr)], accum.at[pl.ds(r0, nr)]), s)
    pltpu.sync_copy(ones_hbm, ones_v)
    plsc.subcore_barrier()

    n_my = jnp.where(wid < EXTRA, BASE_CH + 1, BASE_CH)

    def body(k, carry):
        ch = wid + k * NW
        pltpu.sync_copy(dst_hbm.at[pl.ds(ch * C, C)], idx_v)
        pltpu.sync_copy(ones_v, accum.at[idx_v], add=True)
        return carry

    lax.fori_loop(0, n_my, body, 0)
    plsc.subcore_barrier()
    _copy_rows(lambda r0, nr: pltpu.sync_copy(
        accum.at[pl.ds(r0, nr)], out_hbm.at[c, pl.ds(r0, nr)]), s)


@functools.partial(
    pl.kernel,
    mesh=_mesh,
    out_type=jax.ShapeDtypeStruct((NC, N, D), jnp.float32),
    scratch_types=[
        pltpu.VMEM((C,), jnp.int32),
        pltpu.VMEM((C,), jnp.int32),
        pltpu.VMEM((C, D), jnp.float32),
        pltpu.VMEM_SHARED((N, D), jnp.float32),
        pltpu.SemaphoreType.DMA,
    ],
)
def _spmm_kernel(g_hbm, src_hbm, dst_hbm, zeros_hbm, out_hbm,
                 src_v, dst_v, rows_v, accum, sem):
    c = lax.axis_index("c")
    s = lax.axis_index("s")
    wid = s * NC + c
    _copy_rows(lambda r0, nr: pltpu.sync_copy(
        zeros_hbm.at[pl.ds(r0, nr)], accum.at[pl.ds(r0, nr)]), s)
    plsc.subcore_barrier()

    n_my = jnp.where(wid < EXTRA, BASE_CH + 1, BASE_CH)

    def body(k, carry):
        ch = wid + k * NW
        pltpu.sync_copy(src_hbm.at[pl.ds(ch * C, C)], src_v)
        pltpu.sync_copy(dst_hbm.at[pl.ds(ch * C, C)], dst_v)
        pltpu.async_copy(g_hbm.at[src_v], rows_v, sem).wait()
        pltpu.sync_copy(rows_v, accum.at[dst_v], add=True)
        return carry

    lax.fori_loop(0, n_my, body, 0)
    plsc.subcore_barrier()
    _copy_rows(lambda r0, nr: pltpu.sync_copy(
        accum.at[pl.ds(r0, nr)], out_hbm.at[c, pl.ds(r0, nr)]), s)


NB = 1000
GRID = N // NB


def _first_body(p_ref, x_ref, w_ref, g_ref, u_ref):
    p = p_ref[...]
    deg = p[0, :, :1] + p[1, :, :1] + 1.0
    u = lax.rsqrt(deg)
    u_ref[...] = jnp.broadcast_to(u, (NB, 16))
    h = jnp.dot(x_ref[...], w_ref[...], preferred_element_type=jnp.float32,
                precision=lax.Precision.HIGHEST)
    g_ref[...] = h * u


def _mid_body(s_ref, g_ref, u_ref, b_ref, w_ref, o_ref):
    sv = s_ref[...]
    u = u_ref[...][:, :1]
    t = (sv[0] + sv[1] + g_ref[...]) * u + b_ref[...]
    z = jnp.maximum(t, 0.0)
    o_ref[...] = jnp.dot(z, w_ref[...], preferred_element_type=jnp.float32,
                         precision=lax.Precision.HIGHEST) * u


def _last_body(s_ref, g_ref, u_ref, b_ref, o_ref):
    sv = s_ref[...]
    u = u_ref[...][:, :1]
    o_ref[...] = (sv[0] + sv[1] + g_ref[...]) * u + b_ref[...]


_spec_p = pl.BlockSpec((2, NB, D), lambda i: (0, i, 0))
_spec_x = pl.BlockSpec((NB, D), lambda i: (i, 0))
_spec_w = pl.BlockSpec((D, D), lambda i: (0, 0))
_spec_s = pl.BlockSpec((2, NB, D), lambda i: (0, i, 0))
_spec_u = pl.BlockSpec((NB, 16), lambda i: (i, 0))
_spec_b = pl.BlockSpec((1, D), lambda i: (0, 0))

_first_tc = pl.pallas_call(
    _first_body,
    grid=(GRID,),
    in_specs=[_spec_p, _spec_x, _spec_w],
    out_specs=[_spec_x, _spec_u],
    out_shape=[jax.ShapeDtypeStruct((N, D), jnp.float32),
               jax.ShapeDtypeStruct((N, 16), jnp.float32)],
)

_mid_tc = pl.pallas_call(
    _mid_body,
    grid=(GRID,),
    in_specs=[_spec_s, _spec_x, _spec_u, _spec_b, _spec_w],
    out_specs=_spec_x,
    out_shape=jax.ShapeDtypeStruct((N, D), jnp.float32),
)

_last_tc = pl.pallas_call(
    _last_body,
    grid=(GRID,),
    in_specs=[_spec_s, _spec_x, _spec_u, _spec_b],
    out_specs=_spec_x,
    out_shape=jax.ShapeDtypeStruct((N, D), jnp.float32),
)


@jax.jit
def kernel(x, adj_t, W1, b1, W2, b2, W3, b3):
    adj = adj_t.astype(jnp.int32)
    src = adj[0]
    dst = adj[1]
    onesCD = jnp.ones((C, D), jnp.float32)
    zerosND = jnp.zeros((N, D), jnp.float32)

    p = _deg_kernel(dst, onesCD, zerosND)
    g1, u16 = _first_tc(p, x, W1)
    s1 = _spmm_kernel(g1, src, dst, zerosND)
    g2 = _mid_tc(s1, g1, u16, b1.reshape(1, D), W2)
    s2 = _spmm_kernel(g2, src, dst, zerosND)
    g3 = _mid_tc(s2, g2, u16, b2.reshape(1, D), W3)
    s3 = _spmm_kernel(g3, src, dst, zerosND)
    out = _last_tc(s3, g3, u16, b3.reshape(1, D))
    return out
